# baseline (speedup 1.0000x reference)
import functools

import jax
import jax.numpy as jnp
from jax import lax
from jax.experimental import pallas as pl
from jax.experimental.pallas import tpu as pltpu
from jax.experimental.pallas import tpu_sc as plsc

_EMB = 64
_BATCH = 16384
_B_TOT = 3 * _BATCH
_NC, _NS = 2, 16
_NW = _NC * _NS
_B_PER_W = _B_TOT // _NW
_CHUNK = 128
_N_CHUNK = _B_PER_W // _CHUNK
_REG_W = 1e-5


def _sc_gather(table, idx):
    mesh = plsc.VectorSubcoreMesh(core_axis_name="c", subcore_axis_name="s")

    @functools.partial(
        pl.kernel,
        mesh=mesh,
        compiler_params=pltpu.CompilerParams(use_tc_tiling_on_sc=True),
        out_type=jax.ShapeDtypeStruct((_B_TOT, _EMB), jnp.float32),
        scratch_types=[
            pltpu.VMEM((_B_PER_W,), jnp.int32),
            pltpu.SemaphoreType.DMA,
            pltpu.SemaphoreType.DMA,
        ],
    )
    def gather_kernel(table_hbm, idx_hbm, out_hbm, idx_v, sem_i, sem_g):
        wid = lax.axis_index("s") * _NC + lax.axis_index("c")
        base = wid * _B_PER_W
        pltpu.async_copy(idx_hbm.at[pl.ds(base, _B_PER_W)], idx_v, sem_i).wait()

        @pl.loop(0, _B_PER_W, step=16)
        def _(g):
            vec = idx_v[pl.ds(g, 16)]
            for j in range(16):
                pltpu.async_copy(
                    table_hbm.at[pl.ds(vec[j], 1)],
                    out_hbm.at[pl.ds(base + g + j, 1)],
                    sem_g,
                )

        pltpu.make_async_copy(
            table_hbm.at[pl.ds(0, _B_PER_W)],
            out_hbm.at[pl.ds(base, _B_PER_W)],
            sem_g,
        ).wait()

    return gather_kernel(table, idx)


def _tc_body(g_ref, reward_ref, bpr_ref, reg_ref, loss_ref):
    u = g_ref[0:_BATCH, :]
    p = g_ref[_BATCH:2 * _BATCH, :]
    n = g_ref[2 * _BATCH:3 * _BATCH, :]
    pos_s = jnp.sum(u * p, axis=1)
    neg_s = jnp.sum(u * n, axis=1)
    ij = jnp.sum(p * n, axis=1)
    reward_ref[...] = neg_s + ij
    x = pos_s - neg_s
    bpr = -jnp.mean(jnp.log(jax.nn.sigmoid(x)))
    reg = _REG_W * 0.5 * (jnp.sum(u * u) + jnp.sum(p * p) + jnp.sum(n * n))
    bpr_ref[...] = jnp.full((1, 1), bpr, dtype=jnp.float32)
    reg_ref[...] = jnp.full((1, 1), reg, dtype=jnp.float32)
    loss_ref[...] = jnp.full((1, 1), bpr + reg, dtype=jnp.float32)


def _tc_compute(g):
    return pl.pallas_call(
        _tc_body,
        out_shape=[
            jax.ShapeDtypeStruct((_BATCH,), jnp.float32),
            jax.ShapeDtypeStruct((1, 1), jnp.float32),
            jax.ShapeDtypeStruct((1, 1), jnp.float32),
            jax.ShapeDtypeStruct((1, 1), jnp.float32),
        ],
    )(g)


def kernel(all_embed, u_id, pos_i_id, neg_i_id):
    idx = jnp.concatenate([u_id, pos_i_id, neg_i_id]).astype(jnp.int32)
    g = _sc_gather(all_embed, idx)
    reward, bpr, reg, loss = _tc_compute(g)
    return reward, loss[0, 0], bpr[0, 0], reg[0, 0]


# speedup vs baseline: 1.3398x; 1.3398x over previous
import functools

import jax
import jax.numpy as jnp
from jax import lax
from jax.experimental import pallas as pl
from jax.experimental.pallas import tpu as pltpu
from jax.experimental.pallas import tpu_sc as plsc

_EMB = 64
_PAD = 128
_BATCH = 16384
_B_TOT = 3 * _BATCH
_NC, _NS = 2, 16
_NW = _NC * _NS
_B_PER_W = _B_TOT // _NW
_HALF = _B_PER_W // 2
_CHUNK = 128
_N_CHUNK = _HALF // _CHUNK
_REG_W = 1e-5


def _sc_gather(table128, idx):
    mesh = plsc.VectorSubcoreMesh(core_axis_name="c", subcore_axis_name="s")

    @functools.partial(
        pl.kernel,
        mesh=mesh,
        compiler_params=pltpu.CompilerParams(use_tc_tiling_on_sc=True),
        out_type=jax.ShapeDtypeStruct((_B_TOT, _PAD), jnp.float32),
        scratch_types=[
            pltpu.VMEM((_B_PER_W,), jnp.int32),
            pltpu.VMEM((_HALF, _PAD), jnp.float32),
            pltpu.SemaphoreType.DMA,
            pltpu.SemaphoreType.DMA,
        ],
    )
    def gather_kernel(table_hbm, idx_hbm, out_hbm, idx_v, rows_v, sem_i, sem_g):
        wid = lax.axis_index("s") * _NC + lax.axis_index("c")
        base = wid * _B_PER_W
        pltpu.async_copy(idx_hbm.at[pl.ds(base, _B_PER_W)], idx_v, sem_i).wait()
        for h in range(2):
            copies = []
            for c in range(_N_CHUNK):
                o = h * _HALF + c * _CHUNK
                copies.append(
                    pltpu.async_copy(
                        table_hbm.at[idx_v.at[pl.ds(o, _CHUNK)]],
                        rows_v.at[pl.ds(c * _CHUNK, _CHUNK)],
                        sem_g,
                    )
                )
            for cp in copies:
                cp.wait()
            pltpu.sync_copy(rows_v, out_hbm.at[pl.ds(base + h * _HALF, _HALF)])

    return gather_kernel(table128, idx)


def _tc_body(g_ref, reward_ref, bpr_ref, reg_ref, loss_ref):
    u = g_ref[0]
    p = g_ref[1]
    n = g_ref[2]
    pos_s = jnp.sum(u * p, axis=2)
    neg_s = jnp.sum(u * n, axis=2)
    ij = jnp.sum(p * n, axis=2)
    reward_ref[...] = neg_s + ij
    x = pos_s - neg_s
    bpr = -jnp.mean(jnp.log(jax.nn.sigmoid(x)))
    reg = _REG_W * 0.5 * (jnp.sum(u * u) + jnp.sum(p * p) + jnp.sum(n * n))
    bpr_ref[...] = jnp.full((1, 1), bpr, dtype=jnp.float32)
    reg_ref[...] = jnp.full((1, 1), reg, dtype=jnp.float32)
    loss_ref[...] = jnp.full((1, 1), bpr + reg, dtype=jnp.float32)


def _tc_compute(g4):
    return pl.pallas_call(
        _tc_body,
        out_shape=[
            jax.ShapeDtypeStruct((128, 128), jnp.float32),
            jax.ShapeDtypeStruct((1, 1), jnp.float32),
            jax.ShapeDtypeStruct((1, 1), jnp.float32),
            jax.ShapeDtypeStruct((1, 1), jnp.float32),
        ],
    )(g4)


def kernel(all_embed, u_id, pos_i_id, neg_i_id):
    table128 = jnp.pad(all_embed, ((0, 0), (0, _PAD - _EMB)))
    idx = jnp.concatenate([u_id, pos_i_id, neg_i_id]).astype(jnp.int32)
    g = _sc_gather(table128, idx)
    g4 = g.reshape(3, 128, 128, _PAD)
    reward, bpr, reg, loss = _tc_compute(g4)
    return reward.reshape(_BATCH), loss[0, 0], bpr[0, 0], reg[0, 0]
